# baseline (speedup 1.0000x reference)
import functools

import jax
import jax.numpy as jnp
from jax import lax
from jax.experimental import pallas as pl
from jax.experimental.pallas import tpu as pltpu
from jax.experimental.pallas import tpu_sc as plsc

VOCAB = 100000
D = 64
BATCH = 4096
SEQ = 50

_INFO = plsc.get_sparse_core_info()
NC = _INFO.num_cores
NS = _INFO.num_subcores
NW = NC * NS

BPW = BATCH // NW
LANES = 16
TP = BPW + 9


def _sc_gather(xT, table_real, table_imag):
    mesh = plsc.VectorSubcoreMesh(core_axis_name="c", subcore_axis_name="s")

    @functools.partial(
        pl.kernel,
        mesh=mesh,
        out_type=[
            jax.ShapeDtypeStruct((SEQ, D // 8, BATCH // BPW, 8, BPW), jnp.float32),
            jax.ShapeDtypeStruct((SEQ, D // 8, BATCH // BPW, 8, BPW), jnp.float32),
        ],
        scratch_types=[
            pltpu.VMEM((SEQ, BPW), jnp.int32),
            pltpu.VMEM((2, BPW, D), jnp.float32),
            pltpu.VMEM((2, BPW, D), jnp.float32),
            pltpu.VMEM((2, D // 8, 8, TP), jnp.float32),
            pltpu.VMEM((2, D // 8, 8, TP), jnp.float32),
            pltpu.SemaphoreType.DMA,
            pltpu.SemaphoreType.DMA,
            pltpu.SemaphoreType.DMA,
            pltpu.SemaphoreType.DMA,
        ],
        compiler_params=pltpu.CompilerParams(
            use_tc_tiling_on_sc=False, needs_layout_passes=False),
    )
    def k(xT_hbm, tr_hbm, ti_hbm, outr_hbm, outi_hbm,
          idx_v, buf_r, buf_i, bufT_r, bufT_i, semg0, semg1, semw0, semw1):
        wid = lax.axis_index("s") * NC + lax.axis_index("c")
        semg = [semg0, semg1]
        semw = [semw0, semw1]
        pltpu.sync_copy(xT_hbm.at[:, pl.ds(wid * BPW, BPW)], idx_v)
        iota = lax.iota(jnp.int32, LANES)
        dhs = [((16 * g + iota) >> 3).astype(jnp.int32) for g in range(D // LANES)]
        dls = [((16 * g + iota) & 7).astype(jnp.int32) for g in range(D // LANES)]

        def g_copies(s, b):
            idx_row = idx_v.at[s]
            return (pltpu.make_async_copy(tr_hbm.at[idx_row], buf_r.at[b], semg[b]),
                    pltpu.make_async_copy(ti_hbm.at[idx_row], buf_i.at[b], semg[b]))

        def w_copies(s, b):
            return (pltpu.make_async_copy(bufT_r.at[b, :, :, pl.ds(0, BPW)],
                                          outr_hbm.at[s, :, wid], semw[b]),
                    pltpu.make_async_copy(bufT_i.at[b, :, :, pl.ds(0, BPW)],
                                          outi_hbm.at[s, :, wid], semw[b]))

        def g_start(s, b):
            for c in g_copies(s, b):
                c.start()

        def g_wait(s, b):
            for c in g_copies(s, b):
                c.wait()

        def w_start(s, b):
            for c in w_copies(s, b):
                c.start()

        def w_wait(s, b):
            for c in w_copies(s, b):
                c.wait()

        g_start(0, 0)

        def rnd(r, _):
            for b in range(2):
                s = 2 * r + b

                @pl.when(s + 1 < SEQ)
                def _pref():
                    g_start(s + 1, 1 - b)

                @pl.when(s >= 2)
                def _drain():
                    w_wait(s - 2, b)

                g_wait(s, b)

                def token(t, _c):
                    t_idx = jnp.full((LANES,), t, jnp.int32)
                    for g in range(D // LANES):
                        vr = buf_r[b, t, pl.ds(g * LANES, LANES)]
                        vi = buf_i[b, t, pl.ds(g * LANES, LANES)]
                        plsc.store_scatter(bufT_r.at[b], [dhs[g], dls[g], t_idx], vr)
                        plsc.store_scatter(bufT_i.at[b], [dhs[g], dls[g], t_idx], vi)
                    return _c

                lax.fori_loop(0, BPW, token, 0, unroll=4)
                w_start(s, b)
            return _

        lax.fori_loop(0, SEQ // 2, rnd, 0, unroll=False)
        w_wait(SEQ - 2, 0)
        w_wait(SEQ - 1, 1)

    return k(xT, table_real, table_imag)


def kernel(x, table_real, table_imag):
    xT = jnp.transpose(x.astype(jnp.int32), (1, 0))
    out_r, out_i = _sc_gather(xT, table_real, table_imag)

    def unfold(o):
        return jnp.transpose(o, (2, 4, 0, 1, 3)).reshape(BATCH, SEQ, D)

    return lax.complex(unfold(out_r), unfold(out_i))
